# Initial kernel scaffold; baseline (speedup 1.0000x reference)
import functools

import jax
import jax.numpy as jnp
import numpy as np
from jax import lax
from jax.experimental import pallas as pl
from jax.experimental.pallas import tpu as pltpu
from jax.experimental.pallas import tpu_sc as plsc

_BN_INV = float(1.0 / np.sqrt(1.0 + 1e-5))

N = 50000
D_IN = 39
EMB = 64
GEO = 16
E_LINK = 800000
E_INT = 400000
E_A2B = 400000
B_EMB = 2 * (E_LINK + E_INT + E_A2B)
B_GEO = 2 * (E_LINK + E_INT)

NW = 32
EPW = B_EMB // NW
GPW = B_GEO // NW
CE = 800
CG = 600

BN = 2000
BE = 2000

OFF_LINK_DST = E_LINK // BE
OFF_INT_SRC = 2 * E_LINK // BE
OFF_INT_DST = (2 * E_LINK + E_INT) // BE
OFF_A2B_SRC = (2 * E_LINK + 2 * E_INT) // BE
OFF_A2B_DST = (2 * E_LINK + 2 * E_INT + E_A2B) // BE

_SEL_S = np.zeros((GEO, 1), np.float32); _SEL_S[3, 0] = 1.0
_SEL_P = np.zeros((GEO, 1), np.float32); _SEL_P[0:3, 0] = -2.0
_SEL_V = np.zeros((GEO, 1), np.float32); _SEL_V[4:7, 0] = 1.0


def _node_kernel(x_ref, xyz_ref, vec_ref,
                 w1_ref, b1_ref, g_ref, bt_ref, w2_ref, b2_ref,
                 nw1_ref, nb1_ref, ng_ref, nbt_ref, nw2_ref, nb2_ref,
                 emb_ref, geo_ref, node_ref):
    xb = x_ref[...]
    h = jnp.maximum(jnp.dot(xb, w1_ref[...],
                            preferred_element_type=jnp.float32) + b1_ref[...], 0.0)
    h = g_ref[...] * (h * _BN_INV) + bt_ref[...]
    e = jnp.maximum(jnp.dot(h, w2_ref[...],
                            preferred_element_type=jnp.float32) + b2_ref[...], 0.0)
    emb_ref[...] = e
    hn = jnp.maximum(jnp.dot(e, nw1_ref[...],
                             preferred_element_type=jnp.float32) + nb1_ref[...], 0.0)
    hn = ng_ref[...] * (hn * _BN_INV) + nbt_ref[...]
    node_ref[...] = jnp.dot(hn, nw2_ref[...],
                            preferred_element_type=jnp.float32) + nb2_ref[...]
    p = xyz_ref[...]
    v = vec_ref[...]
    s = jnp.sum(p * p, axis=1, keepdims=True)
    nrm = jnp.sqrt(jnp.sum(v * v, axis=1, keepdims=True))
    vh = v / jnp.maximum(nrm, 1e-8)
    geo_ref[...] = jnp.concatenate(
        [p, s, vh, jnp.zeros((p.shape[0], GEO - 7), jnp.float32)], axis=1)


def _full(shape):
    return pl.BlockSpec(shape, lambda i: tuple(0 for _ in shape))


def _node_stage(x, xyz, vec, fh_W1, fh_b1, fh_g, fh_bt, fh_W2, fh_b2,
                nd_W1, nd_b1, nd_g, nd_bt, nd_W2, nd_b2):
    grid = (N // BN,)
    return pl.pallas_call(
        _node_kernel,
        grid=grid,
        in_specs=[
            pl.BlockSpec((BN, D_IN), lambda i: (i, 0)),
            pl.BlockSpec((BN, 3), lambda i: (i, 0)),
            pl.BlockSpec((BN, 3), lambda i: (i, 0)),
            _full((D_IN, 256)), _full((256,)), _full((256,)), _full((256,)),
            _full((256, EMB)), _full((EMB,)),
            _full((EMB, 128)), _full((128,)), _full((128,)), _full((128,)),
            _full((128, 16)), _full((16,)),
        ],
        out_specs=[
            pl.BlockSpec((BN, EMB), lambda i: (i, 0)),
            pl.BlockSpec((BN, GEO), lambda i: (i, 0)),
            pl.BlockSpec((BN, 16), lambda i: (i, 0)),
        ],
        out_shape=[
            jax.ShapeDtypeStruct((N, EMB), jnp.float32),
            jax.ShapeDtypeStruct((N, GEO), jnp.float32),
            jax.ShapeDtypeStruct((N, 16), jnp.float32),
        ],
    )(x, xyz, vec, fh_W1, fh_b1, fh_g, fh_bt, fh_W2, fh_b2,
      nd_W1, nd_b1, nd_g, nd_bt, nd_W2, nd_b2)


def _gather_body(emb_hbm, geo_hbm, idx_hbm, eout_hbm, gout_hbm,
                 eidx_v, gidx_v, erows_v, grows_v, sem):
    wid = lax.axis_index("s") * 2 + lax.axis_index("c")
    ebase = wid * EPW
    gbase = wid * GPW

    def eloop(i, carry):
        off = ebase + i * CE
        pltpu.sync_copy(idx_hbm.at[pl.ds(off, CE)], eidx_v)
        pltpu.async_copy(emb_hbm.at[eidx_v], erows_v, sem).wait()
        pltpu.sync_copy(erows_v, eout_hbm.at[pl.ds(off, CE)])
        return carry

    lax.fori_loop(0, EPW // CE, eloop, 0)

    def gloop(i, carry):
        off = gbase + i * CG
        pltpu.sync_copy(idx_hbm.at[pl.ds(off, CG)], gidx_v)
        pltpu.async_copy(geo_hbm.at[gidx_v], grows_v, sem).wait()
        pltpu.sync_copy(grows_v, gout_hbm.at[pl.ds(off, CG)])
        return carry

    lax.fori_loop(0, GPW // CG, gloop, 0)


def _gather_stage(emb, geo, idx_all):
    mesh = plsc.VectorSubcoreMesh(core_axis_name="c", subcore_axis_name="s")
    k = pl.kernel(
        _gather_body,
        out_type=(
            jax.ShapeDtypeStruct((B_EMB, EMB), jnp.float32),
            jax.ShapeDtypeStruct((B_GEO, GEO), jnp.float32),
        ),
        mesh=mesh,
        scratch_types=[
            pltpu.VMEM((CE,), jnp.int32),
            pltpu.VMEM((CG,), jnp.int32),
            pltpu.VMEM((CE, EMB), jnp.float32),
            pltpu.VMEM((CG, GEO), jnp.float32),
            pltpu.SemaphoreType.DMA,
        ],
    )
    return k(emb, geo, idx_all)


def _pair_head_kernel(e0_ref, e1_ref, g0_ref, g1_ref,
                      w1a_ref, w1b_ref, wd_ref, wa_ref, b1_ref,
                      g_ref, bt_ref, w2_ref, b2_ref, out_ref):
    e0 = e0_ref[...]
    e1 = e1_ref[...]
    g0 = g0_ref[...]
    g1 = g1_ref[...]
    m = g0 * g1
    ssum = jnp.dot(g0 + g1, jnp.asarray(_SEL_S),
                   preferred_element_type=jnp.float32)
    d = ssum + jnp.dot(m, jnp.asarray(_SEL_P),
                       preferred_element_type=jnp.float32)
    a = jnp.dot(m, jnp.asarray(_SEL_V), preferred_element_type=jnp.float32)
    pre = (jnp.dot(e0, w1a_ref[...], preferred_element_type=jnp.float32)
           + jnp.dot(e1, w1b_ref[...], preferred_element_type=jnp.float32)
           + d * wd_ref[...] + a * wa_ref[...] + b1_ref[...])
    h = jnp.maximum(pre, 0.0)
    h = g_ref[...] * (h * _BN_INV) + bt_ref[...]
    out_ref[...] = jnp.dot(h, w2_ref[...],
                           preferred_element_type=jnp.float32) + b2_ref[...]


def _pair_head(E, G, n_edges, src_off, dst_off, W1, b1, g, bt, W2, b2,
               out_dim):
    w1a = W1[:EMB]
    w1b = W1[EMB:2 * EMB]
    wd = W1[2 * EMB:2 * EMB + 1]
    wa = W1[2 * EMB + 1:2 * EMB + 2]
    hid = W1.shape[1]
    grid = (n_edges // BE,)
    return pl.pallas_call(
        _pair_head_kernel,
        grid=grid,
        in_specs=[
            pl.BlockSpec((BE, EMB), lambda i: (i + src_off, 0)),
            pl.BlockSpec((BE, EMB), lambda i: (i + dst_off, 0)),
            pl.BlockSpec((BE, GEO), lambda i: (i + src_off, 0)),
            pl.BlockSpec((BE, GEO), lambda i: (i + dst_off, 0)),
            _full((EMB, hid)), _full((EMB, hid)),
            _full((1, hid)), _full((1, hid)), _full((hid,)),
            _full((hid,)), _full((hid,)), _full((hid, out_dim)),
            _full((out_dim,)),
        ],
        out_specs=pl.BlockSpec((BE, out_dim), lambda i: (i, 0)),
        out_shape=jax.ShapeDtypeStruct((n_edges, out_dim), jnp.float32),
    )(E, E, G, G, w1a, w1b, wd, wa, b1, g, bt, W2, b2)


def _a2b_head_kernel(e0_ref, e1_ref, w1a_ref, w1b_ref, b1_ref,
                     g_ref, bt_ref, w2_ref, b2_ref, out_ref):
    pre = (jnp.dot(e0_ref[...], w1a_ref[...], preferred_element_type=jnp.float32)
           + jnp.dot(e1_ref[...], w1b_ref[...], preferred_element_type=jnp.float32)
           + b1_ref[...])
    h = jnp.maximum(pre, 0.0)
    h = g_ref[...] * (h * _BN_INV) + bt_ref[...]
    out_ref[...] = jnp.dot(h, w2_ref[...],
                           preferred_element_type=jnp.float32) + b2_ref[...]


def _a2b_head(E, W1, b1, g, bt, W2, b2):
    w1a = W1[:EMB]
    w1b = W1[EMB:]
    hid = W1.shape[1]
    out_dim = W2.shape[1]
    grid = (E_A2B // BE,)
    return pl.pallas_call(
        _a2b_head_kernel,
        grid=grid,
        in_specs=[
            pl.BlockSpec((BE, EMB), lambda i: (i + OFF_A2B_SRC, 0)),
            pl.BlockSpec((BE, EMB), lambda i: (i + OFF_A2B_DST, 0)),
            _full((EMB, hid)), _full((EMB, hid)), _full((hid,)),
            _full((hid,)), _full((hid,)), _full((hid, out_dim)),
            _full((out_dim,)),
        ],
        out_specs=pl.BlockSpec((BE, out_dim), lambda i: (i, 0)),
        out_shape=jax.ShapeDtypeStruct((E_A2B, out_dim), jnp.float32),
    )(E, E, w1a, w1b, b1, g, bt, W2, b2)


def kernel(x, edge_index, edge_attr, interaction_edge_index_pos,
           interaction_edge_index, xyz_data, vector_data, a2b_index, mask,
           fh_W1, fh_b1, fh_g, fh_bt, fh_W2, fh_b2,
           lc_W1, lc_b1, lc_g, lc_bt, lc_W2, lc_b2,
           ab_W1, ab_b1, ab_g, ab_bt, ab_W2, ab_b2,
           nd_W1, nd_b1, nd_g, nd_bt, nd_W2, nd_b2,
           it_W1, it_b1, it_g, it_bt, it_W2, it_b2):
    emb, geo, node_preds = _node_stage(
        x, xyz_data, vector_data, fh_W1, fh_b1, fh_g, fh_bt, fh_W2, fh_b2,
        nd_W1, nd_b1, nd_g, nd_bt, nd_W2, nd_b2)

    idx_all = jnp.concatenate([
        interaction_edge_index[0], interaction_edge_index[1],
        interaction_edge_index_pos[0], interaction_edge_index_pos[1],
        a2b_index[0], a2b_index[1],
    ]).astype(jnp.int32)

    E, G = _gather_stage(emb, geo, idx_all)

    link_preds = _pair_head(E, G, E_LINK, 0, OFF_LINK_DST,
                            lc_W1, lc_b1, lc_g, lc_bt, lc_W2, lc_b2, 1)
    int_preds = _pair_head(E, G, E_INT, OFF_INT_SRC, OFF_INT_DST,
                           it_W1, it_b1, it_g, it_bt, it_W2, it_b2, 3)
    a2b_preds = _a2b_head(E, ab_W1, ab_b1, ab_g, ab_bt, ab_W2, ab_b2)

    return (link_preds, a2b_preds, node_preds, int_preds)


# speedup vs baseline: 2.0594x; 2.0594x over previous
import functools

import jax
import jax.numpy as jnp
import numpy as np
from jax import lax
from jax.experimental import pallas as pl
from jax.experimental.pallas import tpu as pltpu
from jax.experimental.pallas import tpu_sc as plsc

_BN_INV = float(1.0 / np.sqrt(1.0 + 1e-5))

N = 50000
D_IN = 39
EMB = 64
GEO = 16
E_LINK = 800000
E_INT = 400000
E_A2B = 400000
B_EMB = 2 * (E_LINK + E_INT + E_A2B)
B_GEO = 2 * (E_LINK + E_INT)

NW = 32
EPW = B_EMB // NW
GPW = B_GEO // NW
CE = 800
CG = 600

BN = 2000
BE = 2000

OFF_LINK_DST = E_LINK // BE
OFF_INT_SRC = 2 * E_LINK // BE
OFF_INT_DST = (2 * E_LINK + E_INT) // BE
OFF_A2B_SRC = (2 * E_LINK + 2 * E_INT) // BE
OFF_A2B_DST = (2 * E_LINK + 2 * E_INT + E_A2B) // BE

def _geo_selectors():
    col = lax.broadcasted_iota(jnp.int32, (GEO, 1), 0)
    sel_s = (col == 3).astype(jnp.float32)
    sel_p = jnp.where(col < 3, jnp.float32(-2.0), jnp.float32(0.0))
    sel_v = ((col >= 4) & (col < 7)).astype(jnp.float32)
    return sel_s, sel_p, sel_v


def _node_kernel(x_ref, xyz_ref, vec_ref,
                 w1_ref, b1_ref, g_ref, bt_ref, w2_ref, b2_ref,
                 nw1_ref, nb1_ref, ng_ref, nbt_ref, nw2_ref, nb2_ref,
                 emb_ref, geo_ref, node_ref):
    xb = x_ref[...]
    h = jnp.maximum(jnp.dot(xb, w1_ref[...],
                            preferred_element_type=jnp.float32, precision=jax.lax.Precision.HIGHEST) + b1_ref[...], 0.0)
    h = g_ref[...] * (h * _BN_INV) + bt_ref[...]
    e = jnp.maximum(jnp.dot(h, w2_ref[...],
                            preferred_element_type=jnp.float32, precision=jax.lax.Precision.HIGHEST) + b2_ref[...], 0.0)
    emb_ref[...] = e
    hn = jnp.maximum(jnp.dot(e, nw1_ref[...],
                             preferred_element_type=jnp.float32, precision=jax.lax.Precision.HIGHEST) + nb1_ref[...], 0.0)
    hn = ng_ref[...] * (hn * _BN_INV) + nbt_ref[...]
    node_ref[...] = jnp.dot(hn, nw2_ref[...],
                            preferred_element_type=jnp.float32, precision=jax.lax.Precision.HIGHEST) + nb2_ref[...]
    p = xyz_ref[...]
    v = vec_ref[...]
    s = jnp.sum(p * p, axis=1, keepdims=True)
    nrm = jnp.sqrt(jnp.sum(v * v, axis=1, keepdims=True))
    vh = v / jnp.maximum(nrm, 1e-8)
    geo_ref[...] = jnp.concatenate(
        [p, s, vh, jnp.zeros((p.shape[0], GEO - 7), jnp.float32)], axis=1)


def _full(shape):
    return pl.BlockSpec(shape, lambda i: tuple(0 for _ in shape))


def _node_stage(x, xyz, vec, fh_W1, fh_b1, fh_g, fh_bt, fh_W2, fh_b2,
                nd_W1, nd_b1, nd_g, nd_bt, nd_W2, nd_b2):
    grid = (N // BN,)
    return pl.pallas_call(
        _node_kernel,
        grid=grid,
        in_specs=[
            pl.BlockSpec((BN, D_IN), lambda i: (i, 0)),
            pl.BlockSpec((BN, 3), lambda i: (i, 0)),
            pl.BlockSpec((BN, 3), lambda i: (i, 0)),
            _full((D_IN, 256)), _full((256,)), _full((256,)), _full((256,)),
            _full((256, EMB)), _full((EMB,)),
            _full((EMB, 128)), _full((128,)), _full((128,)), _full((128,)),
            _full((128, 16)), _full((16,)),
        ],
        out_specs=[
            pl.BlockSpec((BN, EMB), lambda i: (i, 0)),
            pl.BlockSpec((BN, GEO), lambda i: (i, 0)),
            pl.BlockSpec((BN, 16), lambda i: (i, 0)),
        ],
        out_shape=[
            jax.ShapeDtypeStruct((N, EMB), jnp.float32),
            jax.ShapeDtypeStruct((N, GEO), jnp.float32),
            jax.ShapeDtypeStruct((N, 16), jnp.float32),
        ],
    )(x, xyz, vec, fh_W1, fh_b1, fh_g, fh_bt, fh_W2, fh_b2,
      nd_W1, nd_b1, nd_g, nd_bt, nd_W2, nd_b2)


def _gather_body(emb_hbm, geo_hbm, idx_hbm, eout_hbm, gout_hbm,
                 eidx_v, gidx_v, erows_v, grows_v, sem):
    wid = lax.axis_index("s") * 2 + lax.axis_index("c")
    ebase = wid * EPW
    gbase = wid * GPW

    def eloop(i, carry):
        off = ebase + i * CE
        pltpu.sync_copy(idx_hbm.at[pl.ds(off, CE)], eidx_v)
        pltpu.async_copy(emb_hbm.at[eidx_v], erows_v, sem).wait()
        pltpu.sync_copy(erows_v, eout_hbm.at[pl.ds(off, CE)])
        return carry

    lax.fori_loop(0, EPW // CE, eloop, 0)

    def gloop(i, carry):
        off = gbase + i * CG
        pltpu.sync_copy(idx_hbm.at[pl.ds(off, CG)], gidx_v)
        pltpu.async_copy(geo_hbm.at[gidx_v], grows_v, sem).wait()
        pltpu.sync_copy(grows_v, gout_hbm.at[pl.ds(off, CG)])
        return carry

    lax.fori_loop(0, GPW // CG, gloop, 0)


def _gather_stage(emb, geo, idx_all):
    mesh = plsc.VectorSubcoreMesh(core_axis_name="c", subcore_axis_name="s")
    k = pl.kernel(
        _gather_body,
        out_type=(
            jax.ShapeDtypeStruct((B_EMB, EMB), jnp.float32),
            jax.ShapeDtypeStruct((B_GEO, GEO), jnp.float32),
        ),
        mesh=mesh,
        compiler_params=pltpu.CompilerParams(use_tc_tiling_on_sc=False),
        scratch_types=[
            pltpu.VMEM((CE,), jnp.int32),
            pltpu.VMEM((CG,), jnp.int32),
            pltpu.VMEM((CE, EMB), jnp.float32),
            pltpu.VMEM((CG, GEO), jnp.float32),
            pltpu.SemaphoreType.DMA,
        ],
    )
    return k(emb, geo, idx_all)


def _pair_head_kernel(e0_ref, e1_ref, g0_ref, g1_ref,
                      w1a_ref, w1b_ref, wd_ref, wa_ref, b1_ref,
                      g_ref, bt_ref, w2_ref, b2_ref, out_ref):
    e0 = e0_ref[...]
    e1 = e1_ref[...]
    g0 = g0_ref[...]
    g1 = g1_ref[...]
    m = g0 * g1
    sel_s, sel_p, sel_v = _geo_selectors()
    ssum = jnp.dot(g0 + g1, sel_s, preferred_element_type=jnp.float32, precision=jax.lax.Precision.HIGHEST)
    d = ssum + jnp.dot(m, sel_p, preferred_element_type=jnp.float32, precision=jax.lax.Precision.HIGHEST)
    a = jnp.dot(m, sel_v, preferred_element_type=jnp.float32, precision=jax.lax.Precision.HIGHEST)
    pre = (jnp.dot(e0, w1a_ref[...], preferred_element_type=jnp.float32, precision=jax.lax.Precision.HIGHEST)
           + jnp.dot(e1, w1b_ref[...], preferred_element_type=jnp.float32, precision=jax.lax.Precision.HIGHEST)
           + d * wd_ref[...] + a * wa_ref[...] + b1_ref[...])
    h = jnp.maximum(pre, 0.0)
    h = g_ref[...] * (h * _BN_INV) + bt_ref[...]
    out_ref[...] = jnp.dot(h, w2_ref[...],
                           preferred_element_type=jnp.float32, precision=jax.lax.Precision.HIGHEST) + b2_ref[...]


def _pair_head(E, G, n_edges, src_off, dst_off, W1, b1, g, bt, W2, b2,
               out_dim):
    w1a = W1[:EMB]
    w1b = W1[EMB:2 * EMB]
    wd = W1[2 * EMB:2 * EMB + 1]
    wa = W1[2 * EMB + 1:2 * EMB + 2]
    hid = W1.shape[1]
    grid = (n_edges // BE,)
    return pl.pallas_call(
        _pair_head_kernel,
        grid=grid,
        in_specs=[
            pl.BlockSpec((BE, EMB), lambda i: (i + src_off, 0)),
            pl.BlockSpec((BE, EMB), lambda i: (i + dst_off, 0)),
            pl.BlockSpec((BE, GEO), lambda i: (i + src_off, 0)),
            pl.BlockSpec((BE, GEO), lambda i: (i + dst_off, 0)),
            _full((EMB, hid)), _full((EMB, hid)),
            _full((1, hid)), _full((1, hid)), _full((hid,)),
            _full((hid,)), _full((hid,)), _full((hid, out_dim)),
            _full((out_dim,)),
        ],
        out_specs=pl.BlockSpec((BE, out_dim), lambda i: (i, 0)),
        out_shape=jax.ShapeDtypeStruct((n_edges, out_dim), jnp.float32),
    )(E, E, G, G, w1a, w1b, wd, wa, b1, g, bt, W2, b2)


def _a2b_head_kernel(e0_ref, e1_ref, w1a_ref, w1b_ref, b1_ref,
                     g_ref, bt_ref, w2_ref, b2_ref, out_ref):
    pre = (jnp.dot(e0_ref[...], w1a_ref[...], preferred_element_type=jnp.float32, precision=jax.lax.Precision.HIGHEST)
           + jnp.dot(e1_ref[...], w1b_ref[...], preferred_element_type=jnp.float32, precision=jax.lax.Precision.HIGHEST)
           + b1_ref[...])
    h = jnp.maximum(pre, 0.0)
    h = g_ref[...] * (h * _BN_INV) + bt_ref[...]
    out_ref[...] = jnp.dot(h, w2_ref[...],
                           preferred_element_type=jnp.float32, precision=jax.lax.Precision.HIGHEST) + b2_ref[...]


def _a2b_head(E, W1, b1, g, bt, W2, b2):
    w1a = W1[:EMB]
    w1b = W1[EMB:]
    hid = W1.shape[1]
    out_dim = W2.shape[1]
    grid = (E_A2B // BE,)
    return pl.pallas_call(
        _a2b_head_kernel,
        grid=grid,
        in_specs=[
            pl.BlockSpec((BE, EMB), lambda i: (i + OFF_A2B_SRC, 0)),
            pl.BlockSpec((BE, EMB), lambda i: (i + OFF_A2B_DST, 0)),
            _full((EMB, hid)), _full((EMB, hid)), _full((hid,)),
            _full((hid,)), _full((hid,)), _full((hid, out_dim)),
            _full((out_dim,)),
        ],
        out_specs=pl.BlockSpec((BE, out_dim), lambda i: (i, 0)),
        out_shape=jax.ShapeDtypeStruct((E_A2B, out_dim), jnp.float32),
    )(E, E, w1a, w1b, b1, g, bt, W2, b2)


def kernel(x, edge_index, edge_attr, interaction_edge_index_pos,
           interaction_edge_index, xyz_data, vector_data, a2b_index, mask,
           fh_W1, fh_b1, fh_g, fh_bt, fh_W2, fh_b2,
           lc_W1, lc_b1, lc_g, lc_bt, lc_W2, lc_b2,
           ab_W1, ab_b1, ab_g, ab_bt, ab_W2, ab_b2,
           nd_W1, nd_b1, nd_g, nd_bt, nd_W2, nd_b2,
           it_W1, it_b1, it_g, it_bt, it_W2, it_b2):
    emb, geo, node_preds = _node_stage(
        x, xyz_data, vector_data, fh_W1, fh_b1, fh_g, fh_bt, fh_W2, fh_b2,
        nd_W1, nd_b1, nd_g, nd_bt, nd_W2, nd_b2)

    idx_all = jnp.concatenate([
        interaction_edge_index[0], interaction_edge_index[1],
        interaction_edge_index_pos[0], interaction_edge_index_pos[1],
        a2b_index[0], a2b_index[1],
    ]).astype(jnp.int32)

    E, G = _gather_stage(emb, geo, idx_all)

    link_preds = _pair_head(E, G, E_LINK, 0, OFF_LINK_DST,
                            lc_W1, lc_b1, lc_g, lc_bt, lc_W2, lc_b2, 1)
    int_preds = _pair_head(E, G, E_INT, OFF_INT_SRC, OFF_INT_DST,
                           it_W1, it_b1, it_g, it_bt, it_W2, it_b2, 3)
    a2b_preds = _a2b_head(E, ab_W1, ab_b1, ab_g, ab_bt, ab_W2, ab_b2)

    return (link_preds, a2b_preds, node_preds, int_preds)


# speedup vs baseline: 2.7071x; 1.3145x over previous
import jax
import jax.numpy as jnp
import numpy as np
from jax import lax
from jax.experimental import pallas as pl
from jax.experimental.pallas import tpu as pltpu
from jax.experimental.pallas import tpu_sc as plsc

_BN_INV = float(1.0 / np.sqrt(1.0 + 1e-5))

N = 50000
D_IN = 39
EMB = 64
ROW = 128
E_LINK = 800000
E_INT = 400000
E_A2B = 400000
B_ALL = 2 * (E_LINK + E_INT + E_A2B)

NW = 32
RPW = B_ALL // NW
CE = 400

BN = 2000
BE = 2000

OFF_LINK_DST = E_LINK // BE
OFF_INT_SRC = 2 * E_LINK // BE
OFF_INT_DST = (2 * E_LINK + E_INT) // BE
OFF_A2B_SRC = (2 * E_LINK + 2 * E_INT) // BE
OFF_A2B_DST = (2 * E_LINK + 2 * E_INT + E_A2B) // BE

_PCOL0, _SCOL, _VCOL0 = EMB, EMB + 3, EMB + 4


def _geo_selectors():
    col = lax.broadcasted_iota(jnp.int32, (ROW, 1), 0)
    sel_p = jnp.where((col >= _PCOL0) & (col < _PCOL0 + 3),
                      jnp.float32(-2.0), jnp.float32(0.0))
    sel_v = ((col >= _VCOL0) & (col < _VCOL0 + 3)).astype(jnp.float32)
    return sel_p, sel_v


def _dot(a, b):
    return jnp.dot(a, b, preferred_element_type=jnp.float32,
                   precision=jax.lax.Precision.HIGHEST)


def _node_kernel(x_ref, xyz_ref, vec_ref,
                 w1_ref, b1_ref, g_ref, bt_ref, w2_ref, b2_ref,
                 nw1_ref, nb1_ref, ng_ref, nbt_ref, nw2_ref, nb2_ref,
                 tab_ref, node_ref):
    xb = x_ref[...]
    h = jnp.maximum(_dot(xb, w1_ref[...]) + b1_ref[...], 0.0)
    h = g_ref[...] * (h * _BN_INV) + bt_ref[...]
    e = jnp.maximum(_dot(h, w2_ref[...]) + b2_ref[...], 0.0)
    hn = jnp.maximum(_dot(e, nw1_ref[...]) + nb1_ref[...], 0.0)
    hn = ng_ref[...] * (hn * _BN_INV) + nbt_ref[...]
    node_ref[...] = _dot(hn, nw2_ref[...]) + nb2_ref[...]
    p = xyz_ref[...]
    v = vec_ref[...]
    s = jnp.sum(p * p, axis=1, keepdims=True)
    nrm = jnp.sqrt(jnp.sum(v * v, axis=1, keepdims=True))
    vh = v / jnp.maximum(nrm, 1e-8)
    tab_ref[...] = jnp.concatenate(
        [e, p, s, vh, jnp.zeros((p.shape[0], ROW - EMB - 7), jnp.float32)],
        axis=1)


def _full(shape):
    return pl.BlockSpec(shape, lambda i: tuple(0 for _ in shape))


def _node_stage(x, xyz, vec, fh_W1, fh_b1, fh_g, fh_bt, fh_W2, fh_b2,
                nd_W1, nd_b1, nd_g, nd_bt, nd_W2, nd_b2):
    grid = (N // BN,)
    return pl.pallas_call(
        _node_kernel,
        grid=grid,
        in_specs=[
            pl.BlockSpec((BN, D_IN), lambda i: (i, 0)),
            pl.BlockSpec((BN, 3), lambda i: (i, 0)),
            pl.BlockSpec((BN, 3), lambda i: (i, 0)),
            _full((D_IN, 256)), _full((256,)), _full((256,)), _full((256,)),
            _full((256, EMB)), _full((EMB,)),
            _full((EMB, 128)), _full((128,)), _full((128,)), _full((128,)),
            _full((128, 16)), _full((16,)),
        ],
        out_specs=[
            pl.BlockSpec((BN, ROW), lambda i: (i, 0)),
            pl.BlockSpec((BN, 16), lambda i: (i, 0)),
        ],
        out_shape=[
            jax.ShapeDtypeStruct((N, ROW), jnp.float32),
            jax.ShapeDtypeStruct((N, 16), jnp.float32),
        ],
    )(x, xyz, vec, fh_W1, fh_b1, fh_g, fh_bt, fh_W2, fh_b2,
      nd_W1, nd_b1, nd_g, nd_bt, nd_W2, nd_b2)


def _gather_body(tab_hbm, idx_hbm, out_hbm, idx_v, rows_v, sem):
    wid = lax.axis_index("s") * 2 + lax.axis_index("c")
    base = wid * RPW

    def loop(i, carry):
        off = base + i * CE
        pltpu.sync_copy(idx_hbm.at[pl.ds(off, CE)], idx_v)
        pltpu.async_copy(tab_hbm.at[idx_v], rows_v, sem).wait()
        pltpu.sync_copy(rows_v, out_hbm.at[pl.ds(off, CE)])
        return carry

    lax.fori_loop(0, RPW // CE, loop, 0)


def _gather_stage(tab, idx_all):
    mesh = plsc.VectorSubcoreMesh(core_axis_name="c", subcore_axis_name="s")
    k = pl.kernel(
        _gather_body,
        out_type=jax.ShapeDtypeStruct((B_ALL, ROW), jnp.float32),
        mesh=mesh,
        scratch_types=[
            pltpu.VMEM((CE,), jnp.int32),
            pltpu.VMEM((CE, ROW), jnp.float32),
            pltpu.SemaphoreType.DMA,
        ],
    )
    return k(tab, idx_all)


def _pair_head_kernel(b0_ref, b1_ref, w1a_ref, w1b_ref, wd_ref, wa_ref,
                      bias1_ref, g_ref, bt_ref, w2_ref, b2_ref, out_ref):
    b0 = b0_ref[...]
    b1 = b1_ref[...]
    m = b0 * b1
    sel_p, sel_v = _geo_selectors()
    dcross = _dot(m, sel_p)
    a = _dot(m, sel_v)
    pre = (_dot(b0, w1a_ref[...]) + _dot(b1, w1b_ref[...])
           + dcross * wd_ref[...] + a * wa_ref[...] + bias1_ref[...])
    h = jnp.maximum(pre, 0.0)
    h = g_ref[...] * (h * _BN_INV) + bt_ref[...]
    out_ref[...] = _dot(h, w2_ref[...]) + b2_ref[...]


def _pad_w1(W1, with_dist):
    hid = W1.shape[1]
    out = jnp.zeros((ROW, hid), jnp.float32)
    if with_dist:
        out = out.at[_SCOL].set(W1[2 * EMB])
    return out


def _pair_head(T, n_edges, src_off, dst_off, W1, b1, g, bt, W2, b2,
               out_dim):
    w1a = _pad_w1(W1, True).at[:EMB].set(W1[:EMB])
    w1b = _pad_w1(W1, True).at[:EMB].set(W1[EMB:2 * EMB])
    wd = W1[2 * EMB:2 * EMB + 1]
    wa = W1[2 * EMB + 1:2 * EMB + 2]
    hid = W1.shape[1]
    grid = (n_edges // BE,)
    return pl.pallas_call(
        _pair_head_kernel,
        grid=grid,
        in_specs=[
            pl.BlockSpec((BE, ROW), lambda i: (i + src_off, 0)),
            pl.BlockSpec((BE, ROW), lambda i: (i + dst_off, 0)),
            _full((ROW, hid)), _full((ROW, hid)),
            _full((1, hid)), _full((1, hid)), _full((hid,)),
            _full((hid,)), _full((hid,)), _full((hid, out_dim)),
            _full((out_dim,)),
        ],
        out_specs=pl.BlockSpec((BE, out_dim), lambda i: (i, 0)),
        out_shape=jax.ShapeDtypeStruct((n_edges, out_dim), jnp.float32),
    )(T, T, w1a, w1b, wd, wa, b1, g, bt, W2, b2)


def _a2b_head_kernel(b0_ref, b1_ref, w1a_ref, w1b_ref, bias1_ref,
                     g_ref, bt_ref, w2_ref, b2_ref, out_ref):
    pre = (_dot(b0_ref[...], w1a_ref[...]) + _dot(b1_ref[...], w1b_ref[...])
           + bias1_ref[...])
    h = jnp.maximum(pre, 0.0)
    h = g_ref[...] * (h * _BN_INV) + bt_ref[...]
    out_ref[...] = _dot(h, w2_ref[...]) + b2_ref[...]


def _a2b_head(T, W1, b1, g, bt, W2, b2):
    w1a = _pad_w1(W1, False).at[:EMB].set(W1[:EMB])
    w1b = _pad_w1(W1, False).at[:EMB].set(W1[EMB:])
    hid = W1.shape[1]
    out_dim = W2.shape[1]
    grid = (E_A2B // BE,)
    return pl.pallas_call(
        _a2b_head_kernel,
        grid=grid,
        in_specs=[
            pl.BlockSpec((BE, ROW), lambda i: (i + OFF_A2B_SRC, 0)),
            pl.BlockSpec((BE, ROW), lambda i: (i + OFF_A2B_DST, 0)),
            _full((ROW, hid)), _full((ROW, hid)), _full((hid,)),
            _full((hid,)), _full((hid,)), _full((hid, out_dim)),
            _full((out_dim,)),
        ],
        out_specs=pl.BlockSpec((BE, out_dim), lambda i: (i, 0)),
        out_shape=jax.ShapeDtypeStruct((E_A2B, out_dim), jnp.float32),
    )(T, T, w1a, w1b, b1, g, bt, W2, b2)


def kernel(x, edge_index, edge_attr, interaction_edge_index_pos,
           interaction_edge_index, xyz_data, vector_data, a2b_index, mask,
           fh_W1, fh_b1, fh_g, fh_bt, fh_W2, fh_b2,
           lc_W1, lc_b1, lc_g, lc_bt, lc_W2, lc_b2,
           ab_W1, ab_b1, ab_g, ab_bt, ab_W2, ab_b2,
           nd_W1, nd_b1, nd_g, nd_bt, nd_W2, nd_b2,
           it_W1, it_b1, it_g, it_bt, it_W2, it_b2):
    tab, node_preds = _node_stage(
        x, xyz_data, vector_data, fh_W1, fh_b1, fh_g, fh_bt, fh_W2, fh_b2,
        nd_W1, nd_b1, nd_g, nd_bt, nd_W2, nd_b2)

    idx_all = jnp.concatenate([
        interaction_edge_index[0], interaction_edge_index[1],
        interaction_edge_index_pos[0], interaction_edge_index_pos[1],
        a2b_index[0], a2b_index[1],
    ]).astype(jnp.int32)

    T = _gather_stage(tab, idx_all)

    link_preds = _pair_head(T, E_LINK, 0, OFF_LINK_DST,
                            lc_W1, lc_b1, lc_g, lc_bt, lc_W2, lc_b2, 1)
    int_preds = _pair_head(T, E_INT, OFF_INT_SRC, OFF_INT_DST,
                           it_W1, it_b1, it_g, it_bt, it_W2, it_b2, 3)
    a2b_preds = _a2b_head(T, ab_W1, ab_b1, ab_g, ab_bt, ab_W2, ab_b2)

    return (link_preds, a2b_preds, node_preds, int_preds)


# speedup vs baseline: 3.2706x; 1.2082x over previous
import jax
import jax.numpy as jnp
import numpy as np
from jax import lax
from jax.experimental import pallas as pl
from jax.experimental.pallas import tpu as pltpu
from jax.experimental.pallas import tpu_sc as plsc

_BN_INV = float(1.0 / np.sqrt(1.0 + 1e-5))

N = 50000
D_IN = 39
EMB = 64
ROW = 128
E_LINK = 800000
E_INT = 400000
E_A2B = 400000
B_ALL = 2 * (E_LINK + E_INT + E_A2B)

NW = 32
CE = 200

BN = 2000
BE = 2000


_PCOL0, _SCOL, _VCOL0 = EMB, EMB + 3, EMB + 4


def _geo_selectors():
    col = lax.broadcasted_iota(jnp.int32, (ROW, 1), 0)
    sel_p = jnp.where((col >= _PCOL0) & (col < _PCOL0 + 3),
                      jnp.float32(-2.0), jnp.float32(0.0))
    sel_v = ((col >= _VCOL0) & (col < _VCOL0 + 3)).astype(jnp.float32)
    return sel_p, sel_v


def _dot(a, b):
    return jnp.dot(a, b, preferred_element_type=jnp.float32,
                   precision=jax.lax.Precision.HIGHEST)


def _node_kernel(x_ref, xyz_ref, vec_ref,
                 w1_ref, b1_ref, g_ref, bt_ref, w2_ref, b2_ref,
                 nw1_ref, nb1_ref, ng_ref, nbt_ref, nw2_ref, nb2_ref,
                 tab_ref, node_ref):
    xb = x_ref[...]
    h = jnp.maximum(_dot(xb, w1_ref[...]) + b1_ref[...], 0.0)
    h = g_ref[...] * (h * _BN_INV) + bt_ref[...]
    e = jnp.maximum(_dot(h, w2_ref[...]) + b2_ref[...], 0.0)
    hn = jnp.maximum(_dot(e, nw1_ref[...]) + nb1_ref[...], 0.0)
    hn = ng_ref[...] * (hn * _BN_INV) + nbt_ref[...]
    node_ref[...] = _dot(hn, nw2_ref[...]) + nb2_ref[...]
    p = xyz_ref[...]
    v = vec_ref[...]
    s = jnp.sum(p * p, axis=1, keepdims=True)
    nrm = jnp.sqrt(jnp.sum(v * v, axis=1, keepdims=True))
    vh = v / jnp.maximum(nrm, 1e-8)
    tab_ref[...] = jnp.concatenate(
        [e, p, s, vh, jnp.zeros((p.shape[0], ROW - EMB - 7), jnp.float32)],
        axis=1)


def _full(shape):
    return pl.BlockSpec(shape, lambda i: tuple(0 for _ in shape))


def _node_stage(x, xyz, vec, fh_W1, fh_b1, fh_g, fh_bt, fh_W2, fh_b2,
                nd_W1, nd_b1, nd_g, nd_bt, nd_W2, nd_b2):
    grid = (N // BN,)
    return pl.pallas_call(
        _node_kernel,
        grid=grid,
        in_specs=[
            pl.BlockSpec((BN, D_IN), lambda i: (i, 0)),
            pl.BlockSpec((BN, 3), lambda i: (i, 0)),
            pl.BlockSpec((BN, 3), lambda i: (i, 0)),
            _full((D_IN, 256)), _full((256,)), _full((256,)), _full((256,)),
            _full((256, EMB)), _full((EMB,)),
            _full((EMB, 128)), _full((128,)), _full((128,)), _full((128,)),
            _full((128, 16)), _full((16,)),
        ],
        out_specs=[
            pl.BlockSpec((BN, ROW), lambda i: (i, 0)),
            pl.BlockSpec((BN, 16), lambda i: (i, 0)),
        ],
        out_shape=[
            jax.ShapeDtypeStruct((N, ROW), jnp.float32),
            jax.ShapeDtypeStruct((N, 16), jnp.float32),
        ],
    )(x, xyz, vec, fh_W1, fh_b1, fh_g, fh_bt, fh_W2, fh_b2,
      nd_W1, nd_b1, nd_g, nd_bt, nd_W2, nd_b2)


def _gather_call(tab, idx_all, row_off, n_rows):
    rpw = n_rows // NW
    npair = rpw // (2 * CE)
    tail = (rpw - npair * 2 * CE) // CE

    def body(tab_hbm, idx_hbm, out_hbm,
             idx0, idx1, rows0, rows1, sem0, sem1, ssem0, ssem1):
        wid = lax.axis_index("s") * 2 + lax.axis_index("c")
        base = wid * rpw

        def pair(i, carry):
            off0 = base + 2 * i * CE
            off1 = off0 + CE
            pltpu.sync_copy(idx_hbm.at[pl.ds(row_off + off0, CE)], idx0)
            g0 = pltpu.async_copy(tab_hbm.at[idx0], rows0, sem0)
            pltpu.sync_copy(idx_hbm.at[pl.ds(row_off + off1, CE)], idx1)
            g1 = pltpu.async_copy(tab_hbm.at[idx1], rows1, sem1)
            g0.wait()
            s0 = pltpu.async_copy(rows0, out_hbm.at[pl.ds(off0, CE)], ssem0)
            g1.wait()
            s1 = pltpu.async_copy(rows1, out_hbm.at[pl.ds(off1, CE)], ssem1)
            s0.wait()
            s1.wait()
            return carry

        lax.fori_loop(0, npair, pair, 0)
        if tail:
            off0 = base + npair * 2 * CE
            pltpu.sync_copy(idx_hbm.at[pl.ds(row_off + off0, CE)], idx0)
            pltpu.async_copy(tab_hbm.at[idx0], rows0, sem0).wait()
            pltpu.sync_copy(rows0, out_hbm.at[pl.ds(off0, CE)])

    mesh = plsc.VectorSubcoreMesh(core_axis_name="c", subcore_axis_name="s")
    k = pl.kernel(
        body,
        out_type=jax.ShapeDtypeStruct((n_rows, ROW), jnp.float32),
        mesh=mesh,
        scratch_types=[
            pltpu.VMEM((CE,), jnp.int32),
            pltpu.VMEM((CE,), jnp.int32),
            pltpu.VMEM((CE, ROW), jnp.float32),
            pltpu.VMEM((CE, ROW), jnp.float32),
            pltpu.SemaphoreType.DMA,
            pltpu.SemaphoreType.DMA,
            pltpu.SemaphoreType.DMA,
            pltpu.SemaphoreType.DMA,
        ],
    )
    return k(tab, idx_all)


def _pair_head_kernel(b0_ref, b1_ref, w1a_ref, w1b_ref, wd_ref, wa_ref,
                      bias1_ref, g_ref, bt_ref, w2_ref, b2_ref, out_ref):
    b0 = b0_ref[...]
    b1 = b1_ref[...]
    m = b0 * b1
    sel_p, sel_v = _geo_selectors()
    dcross = _dot(m, sel_p)
    a = _dot(m, sel_v)
    pre = (_dot(b0, w1a_ref[...]) + _dot(b1, w1b_ref[...])
           + dcross * wd_ref[...] + a * wa_ref[...] + bias1_ref[...])
    h = jnp.maximum(pre, 0.0)
    h = g_ref[...] * (h * _BN_INV) + bt_ref[...]
    out_ref[...] = _dot(h, w2_ref[...]) + b2_ref[...]


def _pad_w1(W1, with_dist):
    hid = W1.shape[1]
    out = jnp.zeros((ROW, hid), jnp.float32)
    if with_dist:
        out = out.at[_SCOL].set(W1[2 * EMB])
    return out


def _pair_head(T, n_edges, src_off, dst_off, W1, b1, g, bt, W2, b2,
               out_dim):
    w1a = _pad_w1(W1, True).at[:EMB].set(W1[:EMB])
    w1b = _pad_w1(W1, True).at[:EMB].set(W1[EMB:2 * EMB])
    wd = W1[2 * EMB:2 * EMB + 1]
    wa = W1[2 * EMB + 1:2 * EMB + 2]
    hid = W1.shape[1]
    grid = (n_edges // BE,)
    return pl.pallas_call(
        _pair_head_kernel,
        grid=grid,
        in_specs=[
            pl.BlockSpec((BE, ROW), lambda i: (i + src_off, 0)),
            pl.BlockSpec((BE, ROW), lambda i: (i + dst_off, 0)),
            _full((ROW, hid)), _full((ROW, hid)),
            _full((1, hid)), _full((1, hid)), _full((hid,)),
            _full((hid,)), _full((hid,)), _full((hid, out_dim)),
            _full((out_dim,)),
        ],
        out_specs=pl.BlockSpec((BE, out_dim), lambda i: (i, 0)),
        out_shape=jax.ShapeDtypeStruct((n_edges, out_dim), jnp.float32),
    )(T, T, w1a, w1b, wd, wa, b1, g, bt, W2, b2)


def _a2b_head_kernel(b0_ref, b1_ref, w1a_ref, w1b_ref, bias1_ref,
                     g_ref, bt_ref, w2_ref, b2_ref, out_ref):
    pre = (_dot(b0_ref[...], w1a_ref[...]) + _dot(b1_ref[...], w1b_ref[...])
           + bias1_ref[...])
    h = jnp.maximum(pre, 0.0)
    h = g_ref[...] * (h * _BN_INV) + bt_ref[...]
    out_ref[...] = _dot(h, w2_ref[...]) + b2_ref[...]


def _a2b_head(T, W1, b1, g, bt, W2, b2):
    w1a = _pad_w1(W1, False).at[:EMB].set(W1[:EMB])
    w1b = _pad_w1(W1, False).at[:EMB].set(W1[EMB:])
    hid = W1.shape[1]
    out_dim = W2.shape[1]
    dst_off = E_A2B // BE
    grid = (E_A2B // BE,)
    return pl.pallas_call(
        _a2b_head_kernel,
        grid=grid,
        in_specs=[
            pl.BlockSpec((BE, ROW), lambda i: (i, 0)),
            pl.BlockSpec((BE, ROW), lambda i: (i + dst_off, 0)),
            _full((ROW, hid)), _full((ROW, hid)), _full((hid,)),
            _full((hid,)), _full((hid,)), _full((hid, out_dim)),
            _full((out_dim,)),
        ],
        out_specs=pl.BlockSpec((BE, out_dim), lambda i: (i, 0)),
        out_shape=jax.ShapeDtypeStruct((E_A2B, out_dim), jnp.float32),
    )(T, T, w1a, w1b, b1, g, bt, W2, b2)


def kernel(x, edge_index, edge_attr, interaction_edge_index_pos,
           interaction_edge_index, xyz_data, vector_data, a2b_index, mask,
           fh_W1, fh_b1, fh_g, fh_bt, fh_W2, fh_b2,
           lc_W1, lc_b1, lc_g, lc_bt, lc_W2, lc_b2,
           ab_W1, ab_b1, ab_g, ab_bt, ab_W2, ab_b2,
           nd_W1, nd_b1, nd_g, nd_bt, nd_W2, nd_b2,
           it_W1, it_b1, it_g, it_bt, it_W2, it_b2):
    tab, node_preds = _node_stage(
        x, xyz_data, vector_data, fh_W1, fh_b1, fh_g, fh_bt, fh_W2, fh_b2,
        nd_W1, nd_b1, nd_g, nd_bt, nd_W2, nd_b2)

    idx_all = jnp.concatenate([
        interaction_edge_index[0], interaction_edge_index[1],
        interaction_edge_index_pos[0], interaction_edge_index_pos[1],
        a2b_index[0], a2b_index[1],
    ]).astype(jnp.int32)

    T_link = _gather_call(tab, idx_all, 0, 2 * E_LINK)
    T_int = _gather_call(tab, idx_all, 2 * E_LINK, 2 * E_INT)
    T_a2b = _gather_call(tab, idx_all, 2 * (E_LINK + E_INT), 2 * E_A2B)

    link_preds = _pair_head(T_link, E_LINK, 0, E_LINK // BE,
                            lc_W1, lc_b1, lc_g, lc_bt, lc_W2, lc_b2, 1)
    int_preds = _pair_head(T_int, E_INT, 0, E_INT // BE,
                           it_W1, it_b1, it_g, it_bt, it_W2, it_b2, 3)
    a2b_preds = _a2b_head(T_a2b, ab_W1, ab_b1, ab_g, ab_bt, ab_W2, ab_b2)

    return (link_preds, a2b_preds, node_preds, int_preds)
